# baseline (speedup 1.0000x reference)
import functools

import jax
import jax.numpy as jnp
from jax import lax
from jax.experimental import pallas as pl
from jax.experimental.pallas import tpu as pltpu
from jax.experimental.pallas import tpu_sc as plsc

V = 1000000
D = 32
B = 16384

NC = 2
NS = 16
L = 16
NW = NC * NS
BPW = B // NW
CHUNK = 128
NCHUNK = BPW // CHUNK
WIDE = 128
IPR = D // 2
RPW = WIDE // IPR
GRP = CHUNK // L


def _sc_body(u_hbm, v_hbm, n_hbm, enc_hbm, dec_hbm,
             us_hbm, vs_hbm, ns_hbm,
             iu, iv, inn, wu, wv, wn,
             rows_u, rows_v, rows_n, cu, cv, cn, sem):
    wid = lax.axis_index("s") * NC + lax.axis_index("c")
    rbase = wid * BPW

    pltpu.sync_copy(u_hbm.at[pl.ds(rbase, BPW)], iu)
    pltpu.sync_copy(v_hbm.at[pl.ds(rbase, BPW)], iv)
    pltpu.sync_copy(n_hbm.at[pl.ds(rbase, BPW)], inn)

    for j in range(NCHUNK):
        cb = j * CHUNK
        for s in range(GRP):
            sl = pl.ds(s * L, L)
            csl = pl.ds(cb + s * L, L)
            wu[sl] = iu[csl] >> 3
            wv[sl] = iv[csl] >> 3
            wn[sl] = inn[csl] >> 3
        cp = [pltpu.async_copy(enc_hbm.at[wu], rows_u, sem),
              pltpu.async_copy(dec_hbm.at[wv], rows_v, sem),
              pltpu.async_copy(enc_hbm.at[wn], rows_n, sem)]
        for c in cp:
            c.wait()

        def group(g, carry, cb=cb):
            gb = g * L
            ou = (iu[pl.ds(cb + gb, L)] & (RPW - 1)) * IPR
            ov = (iv[pl.ds(cb + gb, L)] & (RPW - 1)) * IPR
            on = (inn[pl.ds(cb + gb, L)] & (RPW - 1)) * IPR
            for r in range(L):
                row = gb + r
                au = pl.multiple_of(ou[r], IPR)
                av = pl.multiple_of(ov[r], IPR)
                an = pl.multiple_of(on[r], IPR)
                po = pl.multiple_of((cb + row) * IPR, IPR)
                cu[pl.ds(po, IPR)] = rows_u[row, pl.ds(au, IPR)]
                cv[pl.ds(po, IPR)] = rows_v[row, pl.ds(av, IPR)]
                cn[pl.ds(po, IPR)] = rows_n[row, pl.ds(an, IPR)]
            return carry

        lax.fori_loop(0, GRP, group, 0)

    pltpu.sync_copy(cu, us_hbm.at[pl.ds(rbase * IPR, BPW * IPR)])
    pltpu.sync_copy(cv, vs_hbm.at[pl.ds(rbase * IPR, BPW * IPR)])
    pltpu.sync_copy(cn, ns_hbm.at[pl.ds(rbase * IPR, BPW * IPR)])


_sc_gather = functools.partial(
    pl.kernel,
    out_type=(
        jax.ShapeDtypeStruct((B * IPR,), jnp.int32),
        jax.ShapeDtypeStruct((B * IPR,), jnp.int32),
        jax.ShapeDtypeStruct((B * IPR,), jnp.int32),
    ),
    mesh=plsc.VectorSubcoreMesh(core_axis_name="c", subcore_axis_name="s"),
    scratch_types=[
        pltpu.VMEM((BPW,), jnp.int32),
        pltpu.VMEM((BPW,), jnp.int32),
        pltpu.VMEM((BPW,), jnp.int32),
        pltpu.VMEM((CHUNK,), jnp.int32),
        pltpu.VMEM((CHUNK,), jnp.int32),
        pltpu.VMEM((CHUNK,), jnp.int32),
        pltpu.VMEM((CHUNK, WIDE), jnp.int32),
        pltpu.VMEM((CHUNK, WIDE), jnp.int32),
        pltpu.VMEM((CHUNK, WIDE), jnp.int32),
        pltpu.VMEM((BPW * IPR,), jnp.int32),
        pltpu.VMEM((BPW * IPR,), jnp.int32),
        pltpu.VMEM((BPW * IPR,), jnp.int32),
        pltpu.SemaphoreType.DMA,
    ],
)(_sc_body)


def _split_bf16(x_i32):
    even = lax.bitcast_convert_type(x_i32 << 16, jnp.float32)
    odd = lax.bitcast_convert_type(x_i32 & jnp.int32(-65536), jnp.float32)
    return even, odd


def _loss_body(us_ref, vs_ref, ns_ref, out_ref):
    ue, uo = _split_bf16(us_ref[...])
    ve, vo = _split_bf16(vs_ref[...])
    ne, no = _split_bf16(ns_ref[...])
    prod_s = ue * ve + uo * vo
    prod_n = ne * ve + no * vo
    lane = lax.broadcasted_iota(jnp.int32, (WIDE, RPW), 0)
    seg = lax.broadcasted_iota(jnp.int32, (WIDE, RPW), 1)
    m = (lane // IPR == seg).astype(jnp.float32)
    s = jnp.dot(prod_s, m, preferred_element_type=jnp.float32)
    n = -jnp.dot(prod_n, m, preferred_element_type=jnp.float32)
    ls = jnp.minimum(s, 0.0) - jnp.log1p(jnp.exp(-jnp.abs(s)))
    ln = jnp.minimum(n, 0.0) - jnp.log1p(jnp.exp(-jnp.abs(n)))
    out_ref[0, 0] = -jnp.sum(ls + ln)


def kernel(u_pos, u_neg, v_pos, enc_w, dec_w):
    enc_wide = jax.lax.bitcast_convert_type(
        enc_w.astype(jnp.bfloat16).reshape(V, IPR, 2), jnp.int32
    ).reshape(V // RPW, WIDE)
    dec_wide = jax.lax.bitcast_convert_type(
        dec_w.astype(jnp.bfloat16).reshape(V, IPR, 2), jnp.int32
    ).reshape(V // RPW, WIDE)
    us, vs, ns = _sc_gather(u_pos, v_pos, u_neg, enc_wide, dec_wide)
    loss = pl.pallas_call(
        _loss_body,
        out_shape=jax.ShapeDtypeStruct((1, 1), jnp.float32),
        in_specs=[
            pl.BlockSpec(memory_space=pltpu.VMEM),
            pl.BlockSpec(memory_space=pltpu.VMEM),
            pl.BlockSpec(memory_space=pltpu.VMEM),
        ],
        out_specs=pl.BlockSpec(memory_space=pltpu.SMEM),
    )(us.reshape(B // RPW, WIDE), vs.reshape(B // RPW, WIDE),
      ns.reshape(B // RPW, WIDE))
    return loss[0, 0]


# speedup vs baseline: 2.1849x; 2.1849x over previous
import functools

import jax
import jax.numpy as jnp
from jax import lax
from jax.experimental import pallas as pl
from jax.experimental.pallas import tpu as pltpu
from jax.experimental.pallas import tpu_sc as plsc

V = 1000000
D = 32
B = 16384

NC = 2
NS = 16
L = 16
NW = NC * NS
BPW = B // NW
CHUNK = 128
NCHUNK = BPW // CHUNK
WIDE = 128
RPW = WIDE // D
GRP = CHUNK // L


def _sc_body(u_hbm, v_hbm, n_hbm, enc_hbm, dec_hbm,
             ps_hbm, pn_hbm,
             iu, iv, inn, wu, wv, wn,
             rows_u, rows_v, rows_n, part_s, part_n, sem):
    wid = lax.axis_index("s") * NC + lax.axis_index("c")
    rbase = wid * BPW

    pltpu.sync_copy(u_hbm.at[pl.ds(rbase, BPW)], iu)
    pltpu.sync_copy(v_hbm.at[pl.ds(rbase, BPW)], iv)
    pltpu.sync_copy(n_hbm.at[pl.ds(rbase, BPW)], inn)

    for j in range(NCHUNK):
        cb = j * CHUNK
        for s in range(GRP):
            sl = pl.ds(s * L, L)
            csl = pl.ds(cb + s * L, L)
            wu[sl] = iu[csl] >> 2
            wv[sl] = iv[csl] >> 2
            wn[sl] = inn[csl] >> 2
        cp = [pltpu.async_copy(enc_hbm.at[wu], rows_u, sem),
              pltpu.async_copy(dec_hbm.at[wv], rows_v, sem),
              pltpu.async_copy(enc_hbm.at[wn], rows_n, sem)]
        for c in cp:
            c.wait()

        def group(g, carry, cb=cb):
            gb = g * L
            ou = (iu[pl.ds(cb + gb, L)] & (RPW - 1)) * D
            ov = (iv[pl.ds(cb + gb, L)] & (RPW - 1)) * D
            on = (inn[pl.ds(cb + gb, L)] & (RPW - 1)) * D
            for r in range(L):
                row = gb + r
                au, av, an = ou[r], ov[r], on[r]
                ul = rows_u[row, pl.ds(au, L)]
                uh = rows_u[row, pl.ds(au + L, L)]
                vl = rows_v[row, pl.ds(av, L)]
                vh = rows_v[row, pl.ds(av + L, L)]
                nl = rows_n[row, pl.ds(an, L)]
                nh = rows_n[row, pl.ds(an + L, L)]
                part_s[pl.ds((cb + row) * L, L)] = ul * vl + uh * vh
                part_n[pl.ds((cb + row) * L, L)] = nl * vl + nh * vh
            return carry

        lax.fori_loop(0, GRP, group, 0)

    pltpu.sync_copy(part_s, ps_hbm.at[pl.ds(rbase * L, BPW * L)])
    pltpu.sync_copy(part_n, pn_hbm.at[pl.ds(rbase * L, BPW * L)])


_sc_partials = functools.partial(
    pl.kernel,
    out_type=(
        jax.ShapeDtypeStruct((B * L,), jnp.float32),
        jax.ShapeDtypeStruct((B * L,), jnp.float32),
    ),
    mesh=plsc.VectorSubcoreMesh(core_axis_name="c", subcore_axis_name="s"),
    scratch_types=[
        pltpu.VMEM((BPW,), jnp.int32),
        pltpu.VMEM((BPW,), jnp.int32),
        pltpu.VMEM((BPW,), jnp.int32),
        pltpu.VMEM((CHUNK,), jnp.int32),
        pltpu.VMEM((CHUNK,), jnp.int32),
        pltpu.VMEM((CHUNK,), jnp.int32),
        pltpu.VMEM((CHUNK, WIDE), jnp.float32),
        pltpu.VMEM((CHUNK, WIDE), jnp.float32),
        pltpu.VMEM((CHUNK, WIDE), jnp.float32),
        pltpu.VMEM((BPW * L,), jnp.float32),
        pltpu.VMEM((BPW * L,), jnp.float32),
        pltpu.SemaphoreType.DMA,
    ],
)(_sc_body)


def _loss_body(ps_ref, pn_ref, out_ref):
    lane = lax.broadcasted_iota(jnp.int32, (WIDE, 8), 0)
    seg = lax.broadcasted_iota(jnp.int32, (WIDE, 8), 1)
    m = (lane // L == seg).astype(jnp.float32)
    s = jnp.dot(ps_ref[...], m, preferred_element_type=jnp.float32)
    n = -jnp.dot(pn_ref[...], m, preferred_element_type=jnp.float32)
    ls = jnp.minimum(s, 0.0) - jnp.log1p(jnp.exp(-jnp.abs(s)))
    ln = jnp.minimum(n, 0.0) - jnp.log1p(jnp.exp(-jnp.abs(n)))
    out_ref[0, 0] = -jnp.sum(ls + ln)


def kernel(u_pos, u_neg, v_pos, enc_w, dec_w):
    enc_wide = enc_w.reshape(V // RPW, WIDE)
    dec_wide = dec_w.reshape(V // RPW, WIDE)
    part_s, part_n = _sc_partials(u_pos, v_pos, u_neg, enc_wide, dec_wide)
    loss = pl.pallas_call(
        _loss_body,
        out_shape=jax.ShapeDtypeStruct((1, 1), jnp.float32),
        in_specs=[
            pl.BlockSpec(memory_space=pltpu.VMEM),
            pl.BlockSpec(memory_space=pltpu.VMEM),
        ],
        out_specs=pl.BlockSpec(memory_space=pltpu.SMEM),
    )(part_s.reshape(B // 8, WIDE), part_n.reshape(B // 8, WIDE))
    return loss[0, 0]
